# baseline (speedup 1.0000x reference)
import functools

import jax
import jax.numpy as jnp
import numpy as np
from jax import lax
from jax.experimental import pallas as pl
from jax.experimental.pallas import tpu as pltpu
from jax.experimental.pallas import tpu_sc as plsc

_S = 8192

_KEEP_HEX = (
    "3977477ed23beaffedff5dffdd797efff77f5d7fddf797f7ffae9fffd7cefbdfff5b45eb7ffff1fefffb75febef1ef7f"
    "9f776bee77ffaddfa37edb4cf6bada7fffdd1fbefbfedfff5fdea577bbf9fdf37dfb7b79f9f75d7eeef97ff9bf7ef7fc"
    "3ffafffebffdbff5af3dd8bbf67edfadfffbbffed75ca376bbff57ffdf7fbffffdff9dfbeff93d6bedffa7fdf5f6b5ff"
    "3be8f2bdfffefdddbbffbffffffdff7dfd7dbdfbdb5ffffff5aee7a6f3ffe7baaf9fa9fbfdbfc9ffebcecdeddbfaf9ff"
    "bfffffff96bfdadff5adbf3ffffbf7cff50ff7e73ad3f77f7fdfb7effe7f777defedffffe7d3dffefa4fd7dbfffeefff"
    "febd7ff0e7f4fdfeeffe8ffdfc95ff3ffd9fdbf3bf7273fffcfef7bfffff7feffffdf3d9bfc7efe6bf7fffb7ffedffba"
    "f7f6faeffff7debdf17bfedefde3fbb3e75bfff32cfbb5fffbfbb7fff3dfbf3eddeefefbffebf76fcefbbffed5ffffcf"
    "ffffeffffdff6ffd7befdffcfbef1bf4fae6d3bff2ffd5ffbffddffb6bff7f7f3f7dfbf7ffeabefbbffdf7bdbffff77f"
    "9eb923fffd73efdfcfebbfffff7ff1ffffde97edfeef95fe7f39c7ff66effefd7fbffbbffbbffb5f7fffd3fff7f7457f"
    "fffbebbb7fff9ebefeb37e3bfdfdbe7add5ffbbbbfe7ffc71fdef8db9d79ab7ddefdfd3fdef9fbfe1fdff7fffdff7fbf"
    "ffdfbedb3effb7ffa936defe5ffecb6fedee3eb5bef6dffb7dfffbffffeffdfee8fe653ffffbbcdb7fb77fcbef97f7ff"
    "ffbffbebff617ffff7ff73fefbff7fd8f5dddebff7fffffffcff7cbed77f58d8efe35f7bf6f7dfffa7f1fffe47feb9af"
    "44effbbdf7ff9bf777d5defdeefff1fd7bdfeffdaffffbf7df7def7b1ff78feae3f7efed5bf9df75c7f5fdfdfdebfb7f"
    "77fee7dffedf6cf79fffdbfff7ebfdfbdfb7dff97f7fbfff6fbf77fafd7fd7ebeffbddf9e7eebbef67fff77fb6f5bfaf"
    "ff6fbfbd8fbaffffdf7a9f6e7ffbddbff6f7dfefbdfbb7f7deffbfdde7cd75f47ddfbf93dfefffdfdde7bfefdefe73ff"
    "7ffdeffffa87f7f4ffeeffff9fdd2fdf39f7ef7fd3ffeffffffbefcfdfeffbffe37dd7fdc5fffefbbffefff3bfffeb67"
    "ffefff7fffffaf5dff6fdf7e37d77b3efa6afeffdbbd2af9fe5f7dbffdebef5bfee7fa7ffefffeb5fefffdfffff3f1dd"
    "f57ff7fffde7efef77d1f7bbf6dffd7fbfbff7fff7fe9cf8dd7fbe7fbdebbbffaf7fffffc57fbfbf9fffffff59f7b7e8"
    "ff5bfff0bbf7f7a9fbae7fde763dfbfacfffdfff3ffffbdff7f9ff7f7ffbffeff3ffff9bdffffefff3bfbfbff5bda7f6"
    "5ffffb9fffbadb3fef4f877febdfff37f5f6cffffbeff3ffdffddfbedeedfe76bca8dbf4fbfbef7beefedbfc6b57ffbd"
    "ffbf75ffffdedfbfefff97feefdaddaf36dfeef9dfdb677ffa6db379f737ff7fdffdcffbefec7f5ff7da6ed77cf7d0b9"
    "fffb7bfadfbb73dfff7febf6beebefee"
)

_KEEP_NP = np.unpackbits(
    np.frombuffer(bytes.fromhex(_KEEP_HEX), dtype=np.uint8)
).astype(np.float32)

_DROP_IDX = np.where(_KEEP_NP == 0.0)[0].astype(np.int32)
_NDROP_PAD = -(-len(_DROP_IDX) // 16) * 16
_DROP_PAD = np.full((_NDROP_PAD,), _DROP_IDX[-1], dtype=np.int32)
_DROP_PAD[: len(_DROP_IDX)] = _DROP_IDX
_NIDXV = _NDROP_PAD // 16

_ROWS = 4096
_NW = 32
_ROWS_PER_W = _ROWS // _NW
_G = 4
_CHUNK = _G * _S
_NCHUNK = _ROWS_PER_W // _G


def _sc_body(x_hbm, idx_hbm, out_hbm, buf, idxv):
    wid = lax.axis_index("s") * 2 + lax.axis_index("c")
    base = wid * (_ROWS_PER_W * _S)
    pltpu.sync_copy(idx_hbm, idxv)
    zeros = jnp.zeros((16,), jnp.float32)

    def chunk_body(c, carry):
        off = base + c * _CHUNK
        pltpu.sync_copy(x_hbm.at[pl.ds(off, _CHUNK)], buf)
        for g in range(_G):
            rowoff = jnp.int32(g * _S)

            def scat(k, carry2):
                col = idxv[pl.ds(k * 16, 16)]
                plsc.store_scatter(buf, [col + rowoff], zeros)
                return carry2

            lax.fori_loop(0, _NIDXV, scat, 0, unroll=4)
        pltpu.sync_copy(buf, out_hbm.at[pl.ds(off, _CHUNK)])
        return carry

    lax.fori_loop(0, _NCHUNK, chunk_body, 0)


_sc_call = functools.partial(
    pl.kernel,
    mesh=plsc.VectorSubcoreMesh(core_axis_name="c", subcore_axis_name="s"),
    out_type=jax.ShapeDtypeStruct((_ROWS * _S,), jnp.float32),
    scratch_types=[
        pltpu.VMEM((_CHUNK,), jnp.float32),
        pltpu.VMEM((_NDROP_PAD,), jnp.int32),
    ],
    compiler_params=pltpu.CompilerParams(needs_layout_passes=False),
)(_sc_body)


def kernel(x_in):
    B, T, S = x_in.shape
    x_flat = x_in.reshape(B * T * S)
    out = _sc_call(x_flat, jnp.asarray(_DROP_PAD))
    return out.reshape(B, T, S)


# speedup vs baseline: 1.2780x; 1.2780x over previous
import functools

import jax
import jax.numpy as jnp
import numpy as np
from jax import lax
from jax.experimental import pallas as pl
from jax.experimental.pallas import tpu as pltpu
from jax.experimental.pallas import tpu_sc as plsc

_S = 8192

_KEEP_HEX = (
    "3977477ed23beaffedff5dffdd797efff77f5d7fddf797f7ffae9fffd7cefbdfff5b45eb7ffff1fefffb75febef1ef7f"
    "9f776bee77ffaddfa37edb4cf6bada7fffdd1fbefbfedfff5fdea577bbf9fdf37dfb7b79f9f75d7eeef97ff9bf7ef7fc"
    "3ffafffebffdbff5af3dd8bbf67edfadfffbbffed75ca376bbff57ffdf7fbffffdff9dfbeff93d6bedffa7fdf5f6b5ff"
    "3be8f2bdfffefdddbbffbffffffdff7dfd7dbdfbdb5ffffff5aee7a6f3ffe7baaf9fa9fbfdbfc9ffebcecdeddbfaf9ff"
    "bfffffff96bfdadff5adbf3ffffbf7cff50ff7e73ad3f77f7fdfb7effe7f777defedffffe7d3dffefa4fd7dbfffeefff"
    "febd7ff0e7f4fdfeeffe8ffdfc95ff3ffd9fdbf3bf7273fffcfef7bfffff7feffffdf3d9bfc7efe6bf7fffb7ffedffba"
    "f7f6faeffff7debdf17bfedefde3fbb3e75bfff32cfbb5fffbfbb7fff3dfbf3eddeefefbffebf76fcefbbffed5ffffcf"
    "ffffeffffdff6ffd7befdffcfbef1bf4fae6d3bff2ffd5ffbffddffb6bff7f7f3f7dfbf7ffeabefbbffdf7bdbffff77f"
    "9eb923fffd73efdfcfebbfffff7ff1ffffde97edfeef95fe7f39c7ff66effefd7fbffbbffbbffb5f7fffd3fff7f7457f"
    "fffbebbb7fff9ebefeb37e3bfdfdbe7add5ffbbbbfe7ffc71fdef8db9d79ab7ddefdfd3fdef9fbfe1fdff7fffdff7fbf"
    "ffdfbedb3effb7ffa936defe5ffecb6fedee3eb5bef6dffb7dfffbffffeffdfee8fe653ffffbbcdb7fb77fcbef97f7ff"
    "ffbffbebff617ffff7ff73fefbff7fd8f5dddebff7fffffffcff7cbed77f58d8efe35f7bf6f7dfffa7f1fffe47feb9af"
    "44effbbdf7ff9bf777d5defdeefff1fd7bdfeffdaffffbf7df7def7b1ff78feae3f7efed5bf9df75c7f5fdfdfdebfb7f"
    "77fee7dffedf6cf79fffdbfff7ebfdfbdfb7dff97f7fbfff6fbf77fafd7fd7ebeffbddf9e7eebbef67fff77fb6f5bfaf"
    "ff6fbfbd8fbaffffdf7a9f6e7ffbddbff6f7dfefbdfbb7f7deffbfdde7cd75f47ddfbf93dfefffdfdde7bfefdefe73ff"
    "7ffdeffffa87f7f4ffeeffff9fdd2fdf39f7ef7fd3ffeffffffbefcfdfeffbffe37dd7fdc5fffefbbffefff3bfffeb67"
    "ffefff7fffffaf5dff6fdf7e37d77b3efa6afeffdbbd2af9fe5f7dbffdebef5bfee7fa7ffefffeb5fefffdfffff3f1dd"
    "f57ff7fffde7efef77d1f7bbf6dffd7fbfbff7fff7fe9cf8dd7fbe7fbdebbbffaf7fffffc57fbfbf9fffffff59f7b7e8"
    "ff5bfff0bbf7f7a9fbae7fde763dfbfacfffdfff3ffffbdff7f9ff7f7ffbffeff3ffff9bdffffefff3bfbfbff5bda7f6"
    "5ffffb9fffbadb3fef4f877febdfff37f5f6cffffbeff3ffdffddfbedeedfe76bca8dbf4fbfbef7beefedbfc6b57ffbd"
    "ffbf75ffffdedfbfefff97feefdaddaf36dfeef9dfdb677ffa6db379f737ff7fdffdcffbefec7f5ff7da6ed77cf7d0b9"
    "fffb7bfadfbb73dfff7febf6beebefee"
)

_KEEP_NP = np.unpackbits(
    np.frombuffer(bytes.fromhex(_KEEP_HEX), dtype=np.uint8)
).astype(np.float32)

_DROP_IDX = np.where(_KEEP_NP == 0.0)[0].astype(np.int32)
_NDROP_PAD = -(-len(_DROP_IDX) // 16) * 16
_DROP_PAD = np.full((_NDROP_PAD,), _DROP_IDX[-1], dtype=np.int32)
_DROP_PAD[: len(_DROP_IDX)] = _DROP_IDX
_NIDXV = _NDROP_PAD // 16

_ROWS = 4096
_NW = 32
_ROWS_PER_W = _ROWS // _NW
_G = 2
_CHUNK = _G * _S
_NCHUNK = _ROWS_PER_W // _G
_NBUF = 4


def _sc_body(x_hbm, idx_hbm, out_hbm, buf0, buf1, buf2, buf3, idxv, *sems):
    bufs = (buf0, buf1, buf2, buf3)
    sin = sems[:_NBUF]
    sout = sems[_NBUF:]
    wid = lax.axis_index("s") * 2 + lax.axis_index("c")
    base = wid * (_ROWS_PER_W * _S)
    pltpu.sync_copy(idx_hbm, idxv)
    zeros = jnp.zeros((16,), jnp.float32)

    def start_in(t, b):
        pltpu.async_copy(x_hbm.at[pl.ds(base + t * _CHUNK, _CHUNK)], bufs[b], sin[b])

    def start_out(t, b):
        pltpu.async_copy(bufs[b], out_hbm.at[pl.ds(base + t * _CHUNK, _CHUNK)], sout[b])

    def wait_in(b):
        pltpu.make_async_copy(x_hbm.at[pl.ds(base, _CHUNK)], bufs[b], sin[b]).wait()

    def wait_out(b):
        pltpu.make_async_copy(bufs[b], out_hbm.at[pl.ds(base, _CHUNK)], sout[b]).wait()

    start_in(0, 0)
    start_in(1, 1)

    def round_body(c, carry):
        for b0 in range(_NBUF):
            t = c * _NBUF + b0
            bn = (b0 + 2) % _NBUF

            @pl.when(t + 2 < _NCHUNK)
            def _():
                @pl.when(t >= 2)
                def _():
                    wait_out(bn)

                start_in(t + 2, bn)

            wait_in(b0)
            for g in range(_G):
                rowoff = jnp.int32(g * _S)

                def scat(k, carry2):
                    col = idxv[pl.ds(k * 16, 16)]
                    plsc.store_scatter(bufs[b0], [col + rowoff], zeros)
                    return carry2

                lax.fori_loop(0, _NIDXV, scat, 0, unroll=4)
            start_out(t, b0)
        return carry

    lax.fori_loop(0, _NCHUNK // _NBUF, round_body, 0)
    for b0 in range(_NBUF):
        wait_out(b0)


_sc_call = functools.partial(
    pl.kernel,
    mesh=plsc.VectorSubcoreMesh(core_axis_name="c", subcore_axis_name="s"),
    out_type=jax.ShapeDtypeStruct((_ROWS * _S,), jnp.float32),
    scratch_types=[
        pltpu.VMEM((_CHUNK,), jnp.float32),
        pltpu.VMEM((_CHUNK,), jnp.float32),
        pltpu.VMEM((_CHUNK,), jnp.float32),
        pltpu.VMEM((_CHUNK,), jnp.float32),
        pltpu.VMEM((_NDROP_PAD,), jnp.int32),
    ]
    + [pltpu.SemaphoreType.DMA] * (2 * _NBUF),
    compiler_params=pltpu.CompilerParams(needs_layout_passes=False),
)(_sc_body)


def kernel(x_in):
    B, T, S = x_in.shape
    x_flat = x_in.reshape(B * T * S)
    out = _sc_call(x_flat, jnp.asarray(_DROP_PAD))
    return out.reshape(B, T, S)


# speedup vs baseline: 1.2928x; 1.0116x over previous
import functools

import jax
import jax.numpy as jnp
import numpy as np
from jax import lax
from jax.experimental import pallas as pl
from jax.experimental.pallas import tpu as pltpu
from jax.experimental.pallas import tpu_sc as plsc

_S = 8192

_KEEP_HEX = (
    "3977477ed23beaffedff5dffdd797efff77f5d7fddf797f7ffae9fffd7cefbdfff5b45eb7ffff1fefffb75febef1ef7f"
    "9f776bee77ffaddfa37edb4cf6bada7fffdd1fbefbfedfff5fdea577bbf9fdf37dfb7b79f9f75d7eeef97ff9bf7ef7fc"
    "3ffafffebffdbff5af3dd8bbf67edfadfffbbffed75ca376bbff57ffdf7fbffffdff9dfbeff93d6bedffa7fdf5f6b5ff"
    "3be8f2bdfffefdddbbffbffffffdff7dfd7dbdfbdb5ffffff5aee7a6f3ffe7baaf9fa9fbfdbfc9ffebcecdeddbfaf9ff"
    "bfffffff96bfdadff5adbf3ffffbf7cff50ff7e73ad3f77f7fdfb7effe7f777defedffffe7d3dffefa4fd7dbfffeefff"
    "febd7ff0e7f4fdfeeffe8ffdfc95ff3ffd9fdbf3bf7273fffcfef7bfffff7feffffdf3d9bfc7efe6bf7fffb7ffedffba"
    "f7f6faeffff7debdf17bfedefde3fbb3e75bfff32cfbb5fffbfbb7fff3dfbf3eddeefefbffebf76fcefbbffed5ffffcf"
    "ffffeffffdff6ffd7befdffcfbef1bf4fae6d3bff2ffd5ffbffddffb6bff7f7f3f7dfbf7ffeabefbbffdf7bdbffff77f"
    "9eb923fffd73efdfcfebbfffff7ff1ffffde97edfeef95fe7f39c7ff66effefd7fbffbbffbbffb5f7fffd3fff7f7457f"
    "fffbebbb7fff9ebefeb37e3bfdfdbe7add5ffbbbbfe7ffc71fdef8db9d79ab7ddefdfd3fdef9fbfe1fdff7fffdff7fbf"
    "ffdfbedb3effb7ffa936defe5ffecb6fedee3eb5bef6dffb7dfffbffffeffdfee8fe653ffffbbcdb7fb77fcbef97f7ff"
    "ffbffbebff617ffff7ff73fefbff7fd8f5dddebff7fffffffcff7cbed77f58d8efe35f7bf6f7dfffa7f1fffe47feb9af"
    "44effbbdf7ff9bf777d5defdeefff1fd7bdfeffdaffffbf7df7def7b1ff78feae3f7efed5bf9df75c7f5fdfdfdebfb7f"
    "77fee7dffedf6cf79fffdbfff7ebfdfbdfb7dff97f7fbfff6fbf77fafd7fd7ebeffbddf9e7eebbef67fff77fb6f5bfaf"
    "ff6fbfbd8fbaffffdf7a9f6e7ffbddbff6f7dfefbdfbb7f7deffbfdde7cd75f47ddfbf93dfefffdfdde7bfefdefe73ff"
    "7ffdeffffa87f7f4ffeeffff9fdd2fdf39f7ef7fd3ffeffffffbefcfdfeffbffe37dd7fdc5fffefbbffefff3bfffeb67"
    "ffefff7fffffaf5dff6fdf7e37d77b3efa6afeffdbbd2af9fe5f7dbffdebef5bfee7fa7ffefffeb5fefffdfffff3f1dd"
    "f57ff7fffde7efef77d1f7bbf6dffd7fbfbff7fff7fe9cf8dd7fbe7fbdebbbffaf7fffffc57fbfbf9fffffff59f7b7e8"
    "ff5bfff0bbf7f7a9fbae7fde763dfbfacfffdfff3ffffbdff7f9ff7f7ffbffeff3ffff9bdffffefff3bfbfbff5bda7f6"
    "5ffffb9fffbadb3fef4f877febdfff37f5f6cffffbeff3ffdffddfbedeedfe76bca8dbf4fbfbef7beefedbfc6b57ffbd"
    "ffbf75ffffdedfbfefff97feefdaddaf36dfeef9dfdb677ffa6db379f737ff7fdffdcffbefec7f5ff7da6ed77cf7d0b9"
    "fffb7bfadfbb73dfff7febf6beebefee"
)

_KEEP_NP = np.unpackbits(
    np.frombuffer(bytes.fromhex(_KEEP_HEX), dtype=np.uint8)
).astype(np.float32)

_DROP_IDX = np.where(_KEEP_NP == 0.0)[0].astype(np.int32)
_NDROP_PAD = -(-len(_DROP_IDX) // 16) * 16
_DROP_PAD = np.full((_NDROP_PAD,), _DROP_IDX[-1], dtype=np.int32)
_DROP_PAD[: len(_DROP_IDX)] = _DROP_IDX
_NIDXV = _NDROP_PAD // 16

_ROWS = 4096
_NW = 32
_ROWS_PER_W = _ROWS // _NW
_G = 2
_CHUNK = _G * _S
_NCHUNK = _ROWS_PER_W // _G
_NBUF = 4


def _sc_body(x_hbm, idx_hbm, out_hbm, buf0, buf1, buf2, buf3, idxv, *sems):
    bufs = (buf0, buf1, buf2, buf3)
    sin = sems[:_NBUF]
    sout = sems[_NBUF:]
    wid = lax.axis_index("s") * 2 + lax.axis_index("c")
    base = wid * (_ROWS_PER_W * _S)
    pltpu.sync_copy(idx_hbm, idxv)
    zeros = jnp.zeros((16,), jnp.float32)

    def start_in(t, b):
        pltpu.async_copy(x_hbm.at[pl.ds(base + t * _CHUNK, _CHUNK)], bufs[b], sin[b])

    def start_out(t, b):
        pltpu.async_copy(bufs[b], out_hbm.at[pl.ds(base + t * _CHUNK, _CHUNK)], sout[b])

    def wait_in(b):
        pltpu.make_async_copy(x_hbm.at[pl.ds(base, _CHUNK)], bufs[b], sin[b]).wait()

    def wait_out(b):
        pltpu.make_async_copy(bufs[b], out_hbm.at[pl.ds(base, _CHUNK)], sout[b]).wait()

    start_in(0, 0)
    start_in(1, 1)

    def round_body(c, carry):
        for b0 in range(_NBUF):
            t = c * _NBUF + b0
            bn = (b0 + 2) % _NBUF

            @pl.when(t + 2 < _NCHUNK)
            def _():
                @pl.when(t >= 2)
                def _():
                    wait_out(bn)

                start_in(t + 2, bn)

            wait_in(b0)
            start_out(t, b0)
        return carry

    lax.fori_loop(0, _NCHUNK // _NBUF, round_body, 0)
    for b0 in range(_NBUF):
        wait_out(b0)


_sc_call = functools.partial(
    pl.kernel,
    mesh=plsc.VectorSubcoreMesh(core_axis_name="c", subcore_axis_name="s"),
    out_type=jax.ShapeDtypeStruct((_ROWS * _S,), jnp.float32),
    scratch_types=[
        pltpu.VMEM((_CHUNK,), jnp.float32),
        pltpu.VMEM((_CHUNK,), jnp.float32),
        pltpu.VMEM((_CHUNK,), jnp.float32),
        pltpu.VMEM((_CHUNK,), jnp.float32),
        pltpu.VMEM((_NDROP_PAD,), jnp.int32),
    ]
    + [pltpu.SemaphoreType.DMA] * (2 * _NBUF),
    compiler_params=pltpu.CompilerParams(needs_layout_passes=False),
)(_sc_body)


def kernel(x_in):
    B, T, S = x_in.shape
    x_flat = x_in.reshape(B * T * S)
    out = _sc_call(x_flat, jnp.asarray(_DROP_PAD))
    return out.reshape(B, T, S)


# speedup vs baseline: 5.2902x; 4.0920x over previous
import jax
import jax.numpy as jnp
import numpy as np
from jax.experimental import pallas as pl
from jax.experimental.pallas import tpu as pltpu

_BLOCK_ROWS = 128
_S = 8192

_KEEP_HEX = (
    "3977477ed23beaffedff5dffdd797efff77f5d7fddf797f7ffae9fffd7cefbdfff5b45eb7ffff1fefffb75febef1ef7f"
    "9f776bee77ffaddfa37edb4cf6bada7fffdd1fbefbfedfff5fdea577bbf9fdf37dfb7b79f9f75d7eeef97ff9bf7ef7fc"
    "3ffafffebffdbff5af3dd8bbf67edfadfffbbffed75ca376bbff57ffdf7fbffffdff9dfbeff93d6bedffa7fdf5f6b5ff"
    "3be8f2bdfffefdddbbffbffffffdff7dfd7dbdfbdb5ffffff5aee7a6f3ffe7baaf9fa9fbfdbfc9ffebcecdeddbfaf9ff"
    "bfffffff96bfdadff5adbf3ffffbf7cff50ff7e73ad3f77f7fdfb7effe7f777defedffffe7d3dffefa4fd7dbfffeefff"
    "febd7ff0e7f4fdfeeffe8ffdfc95ff3ffd9fdbf3bf7273fffcfef7bfffff7feffffdf3d9bfc7efe6bf7fffb7ffedffba"
    "f7f6faeffff7debdf17bfedefde3fbb3e75bfff32cfbb5fffbfbb7fff3dfbf3eddeefefbffebf76fcefbbffed5ffffcf"
    "ffffeffffdff6ffd7befdffcfbef1bf4fae6d3bff2ffd5ffbffddffb6bff7f7f3f7dfbf7ffeabefbbffdf7bdbffff77f"
    "9eb923fffd73efdfcfebbfffff7ff1ffffde97edfeef95fe7f39c7ff66effefd7fbffbbffbbffb5f7fffd3fff7f7457f"
    "fffbebbb7fff9ebefeb37e3bfdfdbe7add5ffbbbbfe7ffc71fdef8db9d79ab7ddefdfd3fdef9fbfe1fdff7fffdff7fbf"
    "ffdfbedb3effb7ffa936defe5ffecb6fedee3eb5bef6dffb7dfffbffffeffdfee8fe653ffffbbcdb7fb77fcbef97f7ff"
    "ffbffbebff617ffff7ff73fefbff7fd8f5dddebff7fffffffcff7cbed77f58d8efe35f7bf6f7dfffa7f1fffe47feb9af"
    "44effbbdf7ff9bf777d5defdeefff1fd7bdfeffdaffffbf7df7def7b1ff78feae3f7efed5bf9df75c7f5fdfdfdebfb7f"
    "77fee7dffedf6cf79fffdbfff7ebfdfbdfb7dff97f7fbfff6fbf77fafd7fd7ebeffbddf9e7eebbef67fff77fb6f5bfaf"
    "ff6fbfbd8fbaffffdf7a9f6e7ffbddbff6f7dfefbdfbb7f7deffbfdde7cd75f47ddfbf93dfefffdfdde7bfefdefe73ff"
    "7ffdeffffa87f7f4ffeeffff9fdd2fdf39f7ef7fd3ffeffffffbefcfdfeffbffe37dd7fdc5fffefbbffefff3bfffeb67"
    "ffefff7fffffaf5dff6fdf7e37d77b3efa6afeffdbbd2af9fe5f7dbffdebef5bfee7fa7ffefffeb5fefffdfffff3f1dd"
    "f57ff7fffde7efef77d1f7bbf6dffd7fbfbff7fff7fe9cf8dd7fbe7fbdebbbffaf7fffffc57fbfbf9fffffff59f7b7e8"
    "ff5bfff0bbf7f7a9fbae7fde763dfbfacfffdfff3ffffbdff7f9ff7f7ffbffeff3ffff9bdffffefff3bfbfbff5bda7f6"
    "5ffffb9fffbadb3fef4f877febdfff37f5f6cffffbeff3ffdffddfbedeedfe76bca8dbf4fbfbef7beefedbfc6b57ffbd"
    "ffbf75ffffdedfbfefff97feefdaddaf36dfeef9dfdb677ffa6db379f737ff7fdffdcffbefec7f5ff7da6ed77cf7d0b9"
    "fffb7bfadfbb73dfff7febf6beebefee"
)

_KEEP_NP = (
    np.unpackbits(np.frombuffer(bytes.fromhex(_KEEP_HEX), dtype=np.uint8))
    .astype(np.float32)
    .reshape(1, _S)
)


def _mask_body(x_ref, m_ref, o_ref):
    o_ref[...] = jnp.where(m_ref[...] != 0.0, x_ref[...], 0.0)


def kernel(x_in):
    B, T, S = x_in.shape
    keep_f = jnp.asarray(_KEEP_NP)

    rows = B * T
    x2 = x_in.reshape(rows, S)
    grid = (rows // _BLOCK_ROWS,)
    out = pl.pallas_call(
        _mask_body,
        grid=grid,
        in_specs=[
            pl.BlockSpec((_BLOCK_ROWS, S), lambda i: (i, 0)),
            pl.BlockSpec((1, S), lambda i: (0, 0)),
        ],
        out_specs=pl.BlockSpec((_BLOCK_ROWS, S), lambda i: (i, 0)),
        out_shape=jax.ShapeDtypeStruct((rows, S), x_in.dtype),
        compiler_params=pltpu.CompilerParams(vmem_limit_bytes=64 * 1024 * 1024),
    )(x2, keep_f)
    return out.reshape(B, T, S)


# speedup vs baseline: 5.3781x; 1.0166x over previous
import jax
import jax.numpy as jnp
import numpy as np
from jax.experimental import pallas as pl

_BLOCK_ROWS = 256
_S = 8192

_KEEP_HEX = (
    "3977477ed23beaffedff5dffdd797efff77f5d7fddf797f7ffae9fffd7cefbdfff5b45eb7ffff1fefffb75febef1ef7f"
    "9f776bee77ffaddfa37edb4cf6bada7fffdd1fbefbfedfff5fdea577bbf9fdf37dfb7b79f9f75d7eeef97ff9bf7ef7fc"
    "3ffafffebffdbff5af3dd8bbf67edfadfffbbffed75ca376bbff57ffdf7fbffffdff9dfbeff93d6bedffa7fdf5f6b5ff"
    "3be8f2bdfffefdddbbffbffffffdff7dfd7dbdfbdb5ffffff5aee7a6f3ffe7baaf9fa9fbfdbfc9ffebcecdeddbfaf9ff"
    "bfffffff96bfdadff5adbf3ffffbf7cff50ff7e73ad3f77f7fdfb7effe7f777defedffffe7d3dffefa4fd7dbfffeefff"
    "febd7ff0e7f4fdfeeffe8ffdfc95ff3ffd9fdbf3bf7273fffcfef7bfffff7feffffdf3d9bfc7efe6bf7fffb7ffedffba"
    "f7f6faeffff7debdf17bfedefde3fbb3e75bfff32cfbb5fffbfbb7fff3dfbf3eddeefefbffebf76fcefbbffed5ffffcf"
    "ffffeffffdff6ffd7befdffcfbef1bf4fae6d3bff2ffd5ffbffddffb6bff7f7f3f7dfbf7ffeabefbbffdf7bdbffff77f"
    "9eb923fffd73efdfcfebbfffff7ff1ffffde97edfeef95fe7f39c7ff66effefd7fbffbbffbbffb5f7fffd3fff7f7457f"
    "fffbebbb7fff9ebefeb37e3bfdfdbe7add5ffbbbbfe7ffc71fdef8db9d79ab7ddefdfd3fdef9fbfe1fdff7fffdff7fbf"
    "ffdfbedb3effb7ffa936defe5ffecb6fedee3eb5bef6dffb7dfffbffffeffdfee8fe653ffffbbcdb7fb77fcbef97f7ff"
    "ffbffbebff617ffff7ff73fefbff7fd8f5dddebff7fffffffcff7cbed77f58d8efe35f7bf6f7dfffa7f1fffe47feb9af"
    "44effbbdf7ff9bf777d5defdeefff1fd7bdfeffdaffffbf7df7def7b1ff78feae3f7efed5bf9df75c7f5fdfdfdebfb7f"
    "77fee7dffedf6cf79fffdbfff7ebfdfbdfb7dff97f7fbfff6fbf77fafd7fd7ebeffbddf9e7eebbef67fff77fb6f5bfaf"
    "ff6fbfbd8fbaffffdf7a9f6e7ffbddbff6f7dfefbdfbb7f7deffbfdde7cd75f47ddfbf93dfefffdfdde7bfefdefe73ff"
    "7ffdeffffa87f7f4ffeeffff9fdd2fdf39f7ef7fd3ffeffffffbefcfdfeffbffe37dd7fdc5fffefbbffefff3bfffeb67"
    "ffefff7fffffaf5dff6fdf7e37d77b3efa6afeffdbbd2af9fe5f7dbffdebef5bfee7fa7ffefffeb5fefffdfffff3f1dd"
    "f57ff7fffde7efef77d1f7bbf6dffd7fbfbff7fff7fe9cf8dd7fbe7fbdebbbffaf7fffffc57fbfbf9fffffff59f7b7e8"
    "ff5bfff0bbf7f7a9fbae7fde763dfbfacfffdfff3ffffbdff7f9ff7f7ffbffeff3ffff9bdffffefff3bfbfbff5bda7f6"
    "5ffffb9fffbadb3fef4f877febdfff37f5f6cffffbeff3ffdffddfbedeedfe76bca8dbf4fbfbef7beefedbfc6b57ffbd"
    "ffbf75ffffdedfbfefff97feefdaddaf36dfeef9dfdb677ffa6db379f737ff7fdffdcffbefec7f5ff7da6ed77cf7d0b9"
    "fffb7bfadfbb73dfff7febf6beebefee"
)

_KEEP_NP = (
    np.unpackbits(np.frombuffer(bytes.fromhex(_KEEP_HEX), dtype=np.uint8))
    .astype(np.float32)
    .reshape(1, _S)
)


def _mask_body(x_ref, m_ref, o_ref):
    o_ref[...] = jnp.where(m_ref[...] != 0.0, x_ref[...], 0.0)


def kernel(x_in):
    B, T, S = x_in.shape
    keep_f = jnp.asarray(_KEEP_NP)

    rows = B * T
    x2 = x_in.reshape(rows, S)
    grid = (rows // _BLOCK_ROWS,)
    out = pl.pallas_call(
        _mask_body,
        grid=grid,
        in_specs=[
            pl.BlockSpec((_BLOCK_ROWS, S), lambda i: (i, 0)),
            pl.BlockSpec((1, S), lambda i: (0, 0)),
        ],
        out_specs=pl.BlockSpec((_BLOCK_ROWS, S), lambda i: (i, 0)),
        out_shape=jax.ShapeDtypeStruct((rows, S), x_in.dtype),
    )(x2, keep_f)
    return out.reshape(B, T, S)


# speedup vs baseline: 5.3825x; 1.0008x over previous
import jax
import jax.numpy as jnp
import numpy as np
from jax.experimental import pallas as pl

_BLOCK_ROWS = 256
_S = 8192

_KEEP_HEX = (
    "3977477ed23beaffedff5dffdd797efff77f5d7fddf797f7ffae9fffd7cefbdfff5b45eb7ffff1fefffb75febef1ef7f"
    "9f776bee77ffaddfa37edb4cf6bada7fffdd1fbefbfedfff5fdea577bbf9fdf37dfb7b79f9f75d7eeef97ff9bf7ef7fc"
    "3ffafffebffdbff5af3dd8bbf67edfadfffbbffed75ca376bbff57ffdf7fbffffdff9dfbeff93d6bedffa7fdf5f6b5ff"
    "3be8f2bdfffefdddbbffbffffffdff7dfd7dbdfbdb5ffffff5aee7a6f3ffe7baaf9fa9fbfdbfc9ffebcecdeddbfaf9ff"
    "bfffffff96bfdadff5adbf3ffffbf7cff50ff7e73ad3f77f7fdfb7effe7f777defedffffe7d3dffefa4fd7dbfffeefff"
    "febd7ff0e7f4fdfeeffe8ffdfc95ff3ffd9fdbf3bf7273fffcfef7bfffff7feffffdf3d9bfc7efe6bf7fffb7ffedffba"
    "f7f6faeffff7debdf17bfedefde3fbb3e75bfff32cfbb5fffbfbb7fff3dfbf3eddeefefbffebf76fcefbbffed5ffffcf"
    "ffffeffffdff6ffd7befdffcfbef1bf4fae6d3bff2ffd5ffbffddffb6bff7f7f3f7dfbf7ffeabefbbffdf7bdbffff77f"
    "9eb923fffd73efdfcfebbfffff7ff1ffffde97edfeef95fe7f39c7ff66effefd7fbffbbffbbffb5f7fffd3fff7f7457f"
    "fffbebbb7fff9ebefeb37e3bfdfdbe7add5ffbbbbfe7ffc71fdef8db9d79ab7ddefdfd3fdef9fbfe1fdff7fffdff7fbf"
    "ffdfbedb3effb7ffa936defe5ffecb6fedee3eb5bef6dffb7dfffbffffeffdfee8fe653ffffbbcdb7fb77fcbef97f7ff"
    "ffbffbebff617ffff7ff73fefbff7fd8f5dddebff7fffffffcff7cbed77f58d8efe35f7bf6f7dfffa7f1fffe47feb9af"
    "44effbbdf7ff9bf777d5defdeefff1fd7bdfeffdaffffbf7df7def7b1ff78feae3f7efed5bf9df75c7f5fdfdfdebfb7f"
    "77fee7dffedf6cf79fffdbfff7ebfdfbdfb7dff97f7fbfff6fbf77fafd7fd7ebeffbddf9e7eebbef67fff77fb6f5bfaf"
    "ff6fbfbd8fbaffffdf7a9f6e7ffbddbff6f7dfefbdfbb7f7deffbfdde7cd75f47ddfbf93dfefffdfdde7bfefdefe73ff"
    "7ffdeffffa87f7f4ffeeffff9fdd2fdf39f7ef7fd3ffeffffffbefcfdfeffbffe37dd7fdc5fffefbbffefff3bfffeb67"
    "ffefff7fffffaf5dff6fdf7e37d77b3efa6afeffdbbd2af9fe5f7dbffdebef5bfee7fa7ffefffeb5fefffdfffff3f1dd"
    "f57ff7fffde7efef77d1f7bbf6dffd7fbfbff7fff7fe9cf8dd7fbe7fbdebbbffaf7fffffc57fbfbf9fffffff59f7b7e8"
    "ff5bfff0bbf7f7a9fbae7fde763dfbfacfffdfff3ffffbdff7f9ff7f7ffbffeff3ffff9bdffffefff3bfbfbff5bda7f6"
    "5ffffb9fffbadb3fef4f877febdfff37f5f6cffffbeff3ffdffddfbedeedfe76bca8dbf4fbfbef7beefedbfc6b57ffbd"
    "ffbf75ffffdedfbfefff97feefdaddaf36dfeef9dfdb677ffa6db379f737ff7fdffdcffbefec7f5ff7da6ed77cf7d0b9"
    "fffb7bfadfbb73dfff7febf6beebefee"
)

_KEEP_NP = (
    np.unpackbits(np.frombuffer(bytes.fromhex(_KEEP_HEX), dtype=np.uint8))
    .astype(np.float32)
    .reshape(1, _S)
)


def _mask_body(x_ref, m_ref, o_ref):
    o_ref[...] = x_ref[...] * m_ref[...]


def kernel(x_in):
    B, T, S = x_in.shape
    keep_f = jnp.asarray(_KEEP_NP)

    rows = B * T
    x2 = x_in.reshape(rows, S)
    grid = (rows // _BLOCK_ROWS,)
    out = pl.pallas_call(
        _mask_body,
        grid=grid,
        in_specs=[
            pl.BlockSpec((_BLOCK_ROWS, S), lambda i: (i, 0)),
            pl.BlockSpec((1, S), lambda i: (0, 0)),
        ],
        out_specs=pl.BlockSpec((_BLOCK_ROWS, S), lambda i: (i, 0)),
        out_shape=jax.ShapeDtypeStruct((rows, S), x_in.dtype),
    )(x2, keep_f)
    return out.reshape(B, T, S)
